# baseline (speedup 1.0000x reference)
import functools

import jax
import jax.numpy as jnp
from jax import lax
from jax.experimental import pallas as pl
from jax.experimental.pallas import tpu as pltpu
from jax.experimental.pallas import tpu_sc as plsc

QC = 900 * 91
QCP = 81920
NV = QCP // 16
NB1 = 512
NB2 = 128
CAP = 6144
KOUT = 300
KPAD = 304


def _sigmoid_body(x_ref, o_ref):
    o_ref[...] = jax.nn.sigmoid(x_ref[...])


NROW = QCP // 128


def _probs_tc(flat_pad):
    rows = flat_pad.shape[0]
    return pl.pallas_call(
        _sigmoid_body,
        out_shape=jax.ShapeDtypeStruct((rows, 128), jnp.float32),
        grid=(4,),
        in_specs=[pl.BlockSpec((rows // 4, 128), lambda i: (i, 0))],
        out_specs=pl.BlockSpec((rows // 4, 128), lambda i: (i, 0)),
    )(flat_pad)


def _make_sc_topk():
    mesh = plsc.VectorSubcoreMesh(core_axis_name="c", subcore_axis_name="s")

    @functools.partial(
        pl.kernel,
        out_type=[
            jax.ShapeDtypeStruct((32, KPAD), jnp.float32),
            jax.ShapeDtypeStruct((32, KPAD), jnp.int32),
            jax.ShapeDtypeStruct((32, 4 * KPAD), jnp.float32),
        ],
        mesh=mesh,
        compiler_params=pltpu.CompilerParams(needs_layout_passes=False,
                                             use_tc_tiling_on_sc=True),
        scratch_types=[
            pltpu.VMEM((NROW, 128), jnp.float32),
            pltpu.VMEM((3600,), jnp.float32),
            pltpu.VMEM((16 * NB1,), jnp.int32),
            pltpu.VMEM((NB1 + 16,), jnp.int32),
            pltpu.VMEM((640,), jnp.int32),
            pltpu.VMEM((320,), jnp.int32),
            pltpu.VMEM((CAP + 16,), jnp.int32),
            pltpu.VMEM((KPAD,), jnp.float32),
            pltpu.VMEM((KPAD,), jnp.int32),
            pltpu.VMEM((4 * KPAD,), jnp.float32),
            pltpu.SMEM((KPAD,), jnp.int32),
            pltpu.SMEM((KPAD,), jnp.int32),
        ],
    )
    def sc_topk(prob_hbm, boxes_hbm, scores_hbm, labels_hbm, boxout_hbm,
                data, boxv, hist, tot, selbuf, eqbuf, candbuf,
                st_s, st_l, st_b, skey, sidx):
        w = lax.axis_index("s") * 2 + lax.axis_index("c")
        lane = jnp.arange(16, dtype=jnp.int32)
        zeros_i = jnp.zeros((16,), jnp.int32)
        ones_i = jnp.ones((16,), jnp.int32)

        pltpu.sync_copy(prob_hbm.at[pl.ds(w * NROW, NROW)], data)
        pltpu.sync_copy(boxes_hbm.at[w], boxv)

        def keys_at(i):
            return plsc.bitcast(data[i >> 3, pl.ds((i & 7) * 16, 16)],
                                jnp.uint32)

        def gather_keys(iv):
            pv = plsc.load_gather(data, [iv >> 7, iv & 127])
            return plsc.bitcast(pv, jnp.uint32)

        def zero_hist(nbins):
            @plsc.parallel_loop(0, nbins, unroll=8)
            def _(i):
                hist[pl.ds(i * 16, 16)] = zeros_i

        def find(nbins, target):
            ngr = nbins // 16

            @plsc.parallel_loop(0, ngr, unroll=2)
            def _(g):
                acc = zeros_i
                for l in range(16):
                    acc = acc + hist[pl.ds(l * nbins + g * 16, 16)]
                tot[pl.ds(g * 16, 16)] = acc
            tot[pl.ds(nbins, 16)] = zeros_i

            def s(gg, carry):
                g = ngr - 1 - gg
                tv = tot[pl.ds(g * 16, 16)]
                cs = plsc.cumsum(jnp.flip(tv, 0))
                tot[pl.ds(g * 16, 16)] = jnp.flip(cs, 0) + carry
                return carry + cs[15]
            lax.fori_loop(0, ngr, s, jnp.int32(0))

            def fb(g, best):
                sv = tot[pl.ds(g * 16, 16)]
                cand = jnp.where(sv >= target, g * 16 + lane, -1)
                return jnp.maximum(best, jnp.max(cand))
            b = lax.fori_loop(0, ngr, fb, jnp.int32(-1))
            above = tot[pl.ds(b + 1, 16)][0]
            return b, above

        zero_hist(NB1)
        laneoff1 = lane * NB1

        @plsc.parallel_loop(0, NV, unroll=8)
        def _(i):
            bn = (keys_at(i) >> 21).astype(jnp.int32)
            plsc.addupdate_scatter(hist, [laneoff1 + bn], ones_i)

        b1, ab1 = find(NB1, jnp.int32(KOUT))
        t2 = KOUT - ab1
        b1u = b1.astype(jnp.uint32)
        count_b1 = tot[pl.ds(b1, 16)][0] - ab1

        def coll1(ii, carry):
            cur_hi, cur_c = carry
            for s in range(4):
                i = ii * 4 + s
                top = keys_at(i) >> 21
                idxv = i * 16 + lane
                m_hi = top > b1u
                m_c = (top == b1u) & (jnp.broadcast_to(cur_c, (16,)) < CAP)
                plsc.store_compressed(selbuf.at[pl.ds(cur_hi, 16)], idxv,
                                      mask=m_hi)
                plsc.store_compressed(candbuf.at[pl.ds(cur_c, 16)], idxv,
                                      mask=m_c)
                cur_hi = cur_hi + plsc.all_reduce_population_count(m_hi)[0]
                cur_c = cur_c + plsc.all_reduce_population_count(m_c)[0]
            return cur_hi, cur_c
        c_hi, cur_c = lax.fori_loop(0, NV // 4, coll1,
                                    (jnp.int32(0), jnp.int32(0)))
        candbuf[pl.ds(cur_c, 16)] = jnp.full((16,), QC, jnp.int32)

        laneoff2 = lane * NB2

        def cand_keys(j):
            iv = candbuf[pl.ds(j * 16, 16)]
            return gather_keys(iv), iv

        def mid_small(_):
            ncv = (count_b1 + 15) // 16

            def histo_c(shift, pfx_shift, pfx):
                zero_hist(NB2)

                def hc(j, cc):
                    kv, _ = cand_keys(j)
                    m = (j * 16 + lane) < count_b1
                    if pfx_shift is not None:
                        m = m & ((kv >> pfx_shift) == pfx)
                    bn = ((kv >> shift) & 0x7F).astype(jnp.int32)
                    plsc.addupdate_scatter(hist, [laneoff2 + bn], ones_i,
                                           mask=m)
                    return cc
                lax.fori_loop(0, ncv, hc, 0)

            histo_c(14, None, None)
            b2, ab2 = find(NB2, t2)
            t3 = t2 - ab2
            p14 = (b1u << 7) | b2.astype(jnp.uint32)
            histo_c(7, 14, p14)
            b3, ab3 = find(NB2, t3)
            t4 = t3 - ab3
            p7 = (p14 << 7) | b3.astype(jnp.uint32)
            histo_c(0, 7, p7)
            b4, ab4 = find(NB2, t4)
            thr = (p7 << 7) | b4.astype(jnp.uint32)

            def cl(j, carry):
                cur_gt, cur_eq = carry
                kv, iv = cand_keys(j)
                valid = (j * 16 + lane) < count_b1
                m_gt = valid & (kv > thr)
                m_eq = (valid & (kv == thr)
                        & (jnp.broadcast_to(cur_eq, (16,)) < KPAD))
                plsc.store_compressed(selbuf.at[pl.ds(cur_gt, 16)], iv,
                                      mask=m_gt)
                plsc.store_compressed(eqbuf.at[pl.ds(cur_eq, 16)], iv,
                                      mask=m_eq)
                pg = plsc.all_reduce_population_count(m_gt)[0]
                pe = plsc.all_reduce_population_count(m_eq)[0]
                return cur_gt + pg, cur_eq + pe
            lax.fori_loop(0, ncv, cl, (c_hi, jnp.int32(0)))
            return ab2 + ab3 + ab4

        def mid_big(_):
            def histo_f(shift, pfx_shift, pfx):
                zero_hist(NB2)

                def hf(i, cc):
                    kv = keys_at(i)
                    m = (kv >> pfx_shift) == pfx
                    bn = ((kv >> shift) & 0x7F).astype(jnp.int32)
                    plsc.addupdate_scatter(hist, [laneoff2 + bn], ones_i,
                                           mask=m)
                    return cc
                lax.fori_loop(0, NV, hf, 0)

            histo_f(14, 21, b1u)
            b2, ab2 = find(NB2, t2)
            t3 = t2 - ab2
            p14 = (b1u << 7) | b2.astype(jnp.uint32)
            histo_f(7, 14, p14)
            b3, ab3 = find(NB2, t3)
            t4 = t3 - ab3
            p7 = (p14 << 7) | b3.astype(jnp.uint32)
            histo_f(0, 7, p7)
            b4, ab4 = find(NB2, t4)
            thr = (p7 << 7) | b4.astype(jnp.uint32)

            def cl(i, carry):
                cur_gt, cur_eq = carry
                kv = keys_at(i)
                iv = i * 16 + lane
                in_b1 = (kv >> 21) == b1u
                m_gt = in_b1 & (kv > thr)
                m_eq = (in_b1 & (kv == thr)
                        & (jnp.broadcast_to(cur_eq, (16,)) < KPAD))
                plsc.store_compressed(selbuf.at[pl.ds(cur_gt, 16)], iv,
                                      mask=m_gt)
                plsc.store_compressed(eqbuf.at[pl.ds(cur_eq, 16)], iv,
                                      mask=m_eq)
                pg = plsc.all_reduce_population_count(m_gt)[0]
                pe = plsc.all_reduce_population_count(m_eq)[0]
                return cur_gt + pg, cur_eq + pe
            lax.fori_loop(0, NV, cl, (c_hi, jnp.int32(0)))
            return ab2 + ab3 + ab4

        ab234 = lax.cond(count_b1 <= CAP, mid_small, mid_big, 0)
        c = ab1 + ab234

        @plsc.parallel_loop(0, KPAD // 16, unroll=1)
        def _(t):
            selbuf[pl.ds(c + t * 16, 16)] = eqbuf[pl.ds(t * 16, 16)]
        selbuf[pl.ds(KOUT, 16)] = QC + lane

        @plsc.parallel_loop(0, KPAD // 16, unroll=1)
        def _(t):
            iv = selbuf[pl.ds(t * 16, 16)]
            kv = plsc.bitcast(gather_keys(iv), jnp.int32)
            for l in range(16):
                skey[t * 16 + l] = kv[l]
                sidx[t * 16 + l] = iv[l]

        @plsc.parallel_loop(0, KPAD // 16, unroll=1)
        def _(t):
            qi = selbuf[pl.ds(t * 16, 16)]
            qk = plsc.bitcast(gather_keys(qi), jnp.int32)

            def d(j, r):
                jj = j * 2
                for u in range(2):
                    dk = skey[jj + u]
                    di = sidx[jj + u]
                    beat = (dk > qk) | ((dk == qk) & (di < qi))
                    r = r + beat.astype(jnp.int32)
                return r
            rank = lax.fori_loop(0, KPAD // 2, d, zeros_i)

            score = plsc.bitcast(qk, jnp.float32)
            row = ((qi.astype(jnp.float32) + 0.5)
                   * jnp.float32(1.0 / 91.0)).astype(jnp.int32)
            row = jnp.minimum(row, 899)
            lab = qi - row * 91
            plsc.store_scatter(st_s, [rank], score)
            plsc.store_scatter(st_l, [rank], lab)
            for jb in range(4):
                bv = plsc.load_gather(boxv, [row * 4 + jb])
                plsc.store_scatter(st_b, [jb * KPAD + rank], bv)

        pltpu.sync_copy(st_s, scores_hbm.at[w])
        pltpu.sync_copy(st_l, labels_hbm.at[w])
        pltpu.sync_copy(st_b, boxout_hbm.at[w])

    return sc_topk


_sc_topk = _make_sc_topk()


def kernel(pred_logits, pred_boxes, target_sizes):
    del target_sizes
    b, q, cc = pred_logits.shape
    flat = pred_logits.reshape(b, q * cc)
    flat_pad = jnp.pad(flat, ((0, 0), (0, QCP - QC)),
                       constant_values=jnp.float32(-1e30))
    prob = _probs_tc(flat_pad.reshape(b * NROW, 128))
    boxes_flat = pred_boxes.reshape(b, q * 4)
    scores_p, labels_p, boxes_p = _sc_topk(prob, boxes_flat)
    scores = scores_p[:, :KOUT]
    labels = labels_p[:, :KOUT]
    boxes = boxes_p.reshape(b, 4, KPAD)[:, :, :KOUT].transpose(0, 2, 1)
    return scores, labels, boxes


# speedup vs baseline: 1.3773x; 1.3773x over previous
import functools

import jax
import jax.numpy as jnp
from jax import lax
from jax.experimental import pallas as pl
from jax.experimental.pallas import tpu as pltpu
from jax.experimental.pallas import tpu_sc as plsc

QC = 900 * 91
QCP = 81920
NV = QCP // 16
NB1 = 512
NB2 = 128
CAP = 6144
KOUT = 300
KPAD = 304


def _sigmoid_body(x_ref, o_ref):
    o_ref[...] = jax.nn.sigmoid(x_ref[...])


NROW = QCP // 128


def _probs_tc(flat_pad):
    rows = flat_pad.shape[0]
    return pl.pallas_call(
        _sigmoid_body,
        out_shape=jax.ShapeDtypeStruct((rows, 128), jnp.float32),
        grid=(4,),
        in_specs=[pl.BlockSpec((rows // 4, 128), lambda i: (i, 0))],
        out_specs=pl.BlockSpec((rows // 4, 128), lambda i: (i, 0)),
    )(flat_pad)


def _make_sc_topk():
    mesh = plsc.VectorSubcoreMesh(core_axis_name="c", subcore_axis_name="s")

    @functools.partial(
        pl.kernel,
        out_type=[
            jax.ShapeDtypeStruct((32, KPAD), jnp.float32),
            jax.ShapeDtypeStruct((32, KPAD), jnp.int32),
            jax.ShapeDtypeStruct((32, 4 * KPAD), jnp.float32),
        ],
        mesh=mesh,
        compiler_params=pltpu.CompilerParams(needs_layout_passes=False,
                                             use_tc_tiling_on_sc=True),
        scratch_types=[
            pltpu.VMEM((NROW, 128), jnp.float32),
            pltpu.VMEM((3600,), jnp.float32),
            pltpu.VMEM((16 * NB1,), jnp.int32),
            pltpu.VMEM((NB1 + 16,), jnp.int32),
            pltpu.VMEM((640,), jnp.int32),
            pltpu.VMEM((320,), jnp.int32),
            pltpu.VMEM((CAP + 16,), jnp.int32),
            pltpu.VMEM((KPAD,), jnp.float32),
            pltpu.VMEM((KPAD,), jnp.int32),
            pltpu.VMEM((4 * KPAD,), jnp.float32),
            pltpu.SMEM((KPAD,), jnp.int32),
            pltpu.SMEM((KPAD,), jnp.int32),
        ],
    )
    def sc_topk(prob_hbm, boxes_hbm, scores_hbm, labels_hbm, boxout_hbm,
                data, boxv, hist, tot, selbuf, eqbuf, candbuf,
                st_s, st_l, st_b, skey, sidx):
        w = lax.axis_index("s") * 2 + lax.axis_index("c")
        lane = jnp.arange(16, dtype=jnp.int32)
        zeros_i = jnp.zeros((16,), jnp.int32)
        ones_i = jnp.ones((16,), jnp.int32)

        pltpu.sync_copy(prob_hbm.at[pl.ds(w * NROW, NROW)], data)
        pltpu.sync_copy(boxes_hbm.at[w], boxv)

        def keys_at(i):
            return plsc.bitcast(data[i >> 3, pl.ds((i & 7) * 16, 16)],
                                jnp.uint32)

        def gather_keys(iv):
            pv = plsc.load_gather(data, [iv >> 7, iv & 127])
            return plsc.bitcast(pv, jnp.uint32)

        def zero_hist(nbins):
            @plsc.parallel_loop(0, nbins, unroll=8)
            def _(i):
                hist[pl.ds(i * 16, 16)] = zeros_i

        def find(nbins, target):
            ngr = nbins // 16

            @plsc.parallel_loop(0, ngr, unroll=2)
            def _(g):
                acc = zeros_i
                for l in range(16):
                    acc = acc + hist[pl.ds(l * nbins + g * 16, 16)]
                tot[pl.ds(g * 16, 16)] = acc
            tot[pl.ds(nbins, 16)] = zeros_i

            def s(gg, carry):
                g = ngr - 1 - gg
                tv = tot[pl.ds(g * 16, 16)]
                cs = plsc.cumsum(jnp.flip(tv, 0))
                tot[pl.ds(g * 16, 16)] = jnp.flip(cs, 0) + carry
                return carry + cs[15]
            lax.fori_loop(0, ngr, s, jnp.int32(0))

            def fb(g, best):
                sv = tot[pl.ds(g * 16, 16)]
                cand = jnp.where(sv >= target, g * 16 + lane, -1)
                return jnp.maximum(best, jnp.max(cand))
            b = lax.fori_loop(0, ngr, fb, jnp.int32(-1))
            above = tot[pl.ds(b + 1, 16)][0]
            return b, above

        zero_hist(NB1)
        laneoff1 = lane * NB1

        @plsc.parallel_loop(0, NV, unroll=8)
        def _(i):
            bn = (keys_at(i) >> 21).astype(jnp.int32)
            plsc.addupdate_scatter(hist, [laneoff1 + bn], ones_i)

        b1, ab1 = find(NB1, jnp.int32(KOUT))
        t2 = KOUT - ab1
        b1u = b1.astype(jnp.uint32)
        count_b1 = tot[pl.ds(b1, 16)][0] - ab1

        fifteen = jnp.full((16,), 15, jnp.int32)
        CAPG = CAP - 48

        gdn = lax.GatherDimensionNumbers(offset_dims=(),
                                         collapsed_slice_dims=(0,),
                                         start_index_map=(0,))

        def splat_last(cs):
            return lax.gather(cs, fifteen[:, None], gdn, (1,),
                              mode=lax.GatherScatterMode.PROMISE_IN_BOUNDS)

        def coll1(ii, carry):
            hi_v, c_v = carry
            gate = c_v < CAPG
            csh, csc, mh, mc, iv = [], [], [], [], []
            for s in range(4):
                i = ii * 4 + s
                top = keys_at(i) >> 21
                m_hi = top > b1u
                m_c = (top == b1u) & gate
                csh.append(plsc.cumsum(m_hi.astype(jnp.int32)))
                csc.append(plsc.cumsum(m_c.astype(jnp.int32)))
                mh.append(m_hi)
                mc.append(m_c)
                iv.append(i * 16 + lane)
            for s in range(4):
                plsc.store_scatter(selbuf, [hi_v + csh[s] - 1], iv[s],
                                   mask=mh[s])
                plsc.store_scatter(candbuf, [c_v + csc[s] - 1], iv[s],
                                   mask=mc[s])
                hi_v = hi_v + splat_last(csh[s])
                c_v = c_v + splat_last(csc[s])
            return hi_v, c_v
        hi_v, c_v = lax.fori_loop(0, NV // 4, coll1, (zeros_i, zeros_i))
        c_hi, cur_c = hi_v[0], c_v[0]
        candbuf[pl.ds(cur_c, 16)] = jnp.full((16,), QC, jnp.int32)

        laneoff2 = lane * NB2

        def cand_keys(j):
            iv = candbuf[pl.ds(j * 16, 16)]
            return gather_keys(iv), iv

        def mid_small(_):
            ncv = (count_b1 + 15) // 16

            def histo_c(shift, pfx_shift, pfx):
                zero_hist(NB2)

                def hc(j, cc):
                    kv, _ = cand_keys(j)
                    m = (j * 16 + lane) < count_b1
                    if pfx_shift is not None:
                        m = m & ((kv >> pfx_shift) == pfx)
                    bn = ((kv >> shift) & 0x7F).astype(jnp.int32)
                    plsc.addupdate_scatter(hist, [laneoff2 + bn], ones_i,
                                           mask=m)
                    return cc
                lax.fori_loop(0, ncv, hc, 0)

            histo_c(14, None, None)
            b2, ab2 = find(NB2, t2)
            t3 = t2 - ab2
            p14 = (b1u << 7) | b2.astype(jnp.uint32)
            histo_c(7, 14, p14)
            b3, ab3 = find(NB2, t3)
            t4 = t3 - ab3
            p7 = (p14 << 7) | b3.astype(jnp.uint32)
            histo_c(0, 7, p7)
            b4, ab4 = find(NB2, t4)
            thr = (p7 << 7) | b4.astype(jnp.uint32)

            def cl(j, carry):
                cur_gt, cur_eq = carry
                kv, iv = cand_keys(j)
                valid = (j * 16 + lane) < count_b1
                m_gt = valid & (kv > thr)
                m_eq = (valid & (kv == thr)
                        & (jnp.broadcast_to(cur_eq, (16,)) < KPAD))
                plsc.store_compressed(selbuf.at[pl.ds(cur_gt, 16)], iv,
                                      mask=m_gt)
                plsc.store_compressed(eqbuf.at[pl.ds(cur_eq, 16)], iv,
                                      mask=m_eq)
                pg = plsc.all_reduce_population_count(m_gt)[0]
                pe = plsc.all_reduce_population_count(m_eq)[0]
                return cur_gt + pg, cur_eq + pe
            lax.fori_loop(0, ncv, cl, (c_hi, jnp.int32(0)))
            return ab2 + ab3 + ab4

        def mid_big(_):
            def histo_f(shift, pfx_shift, pfx):
                zero_hist(NB2)

                def hf(i, cc):
                    kv = keys_at(i)
                    m = (kv >> pfx_shift) == pfx
                    bn = ((kv >> shift) & 0x7F).astype(jnp.int32)
                    plsc.addupdate_scatter(hist, [laneoff2 + bn], ones_i,
                                           mask=m)
                    return cc
                lax.fori_loop(0, NV, hf, 0)

            histo_f(14, 21, b1u)
            b2, ab2 = find(NB2, t2)
            t3 = t2 - ab2
            p14 = (b1u << 7) | b2.astype(jnp.uint32)
            histo_f(7, 14, p14)
            b3, ab3 = find(NB2, t3)
            t4 = t3 - ab3
            p7 = (p14 << 7) | b3.astype(jnp.uint32)
            histo_f(0, 7, p7)
            b4, ab4 = find(NB2, t4)
            thr = (p7 << 7) | b4.astype(jnp.uint32)

            def cl(i, carry):
                cur_gt, cur_eq = carry
                kv = keys_at(i)
                iv = i * 16 + lane
                in_b1 = (kv >> 21) == b1u
                m_gt = in_b1 & (kv > thr)
                m_eq = (in_b1 & (kv == thr)
                        & (jnp.broadcast_to(cur_eq, (16,)) < KPAD))
                plsc.store_compressed(selbuf.at[pl.ds(cur_gt, 16)], iv,
                                      mask=m_gt)
                plsc.store_compressed(eqbuf.at[pl.ds(cur_eq, 16)], iv,
                                      mask=m_eq)
                pg = plsc.all_reduce_population_count(m_gt)[0]
                pe = plsc.all_reduce_population_count(m_eq)[0]
                return cur_gt + pg, cur_eq + pe
            lax.fori_loop(0, NV, cl, (c_hi, jnp.int32(0)))
            return ab2 + ab3 + ab4

        ab234 = lax.cond(count_b1 <= CAPG, mid_small, mid_big, 0)
        c = ab1 + ab234

        @plsc.parallel_loop(0, KPAD // 16, unroll=1)
        def _(t):
            selbuf[pl.ds(c + t * 16, 16)] = eqbuf[pl.ds(t * 16, 16)]
        selbuf[pl.ds(KOUT, 16)] = QC + lane

        @plsc.parallel_loop(0, KPAD // 16, unroll=1)
        def _(t):
            iv = selbuf[pl.ds(t * 16, 16)]
            kv = plsc.bitcast(gather_keys(iv), jnp.int32)
            for l in range(16):
                skey[t * 16 + l] = kv[l]
                sidx[t * 16 + l] = iv[l]

        @plsc.parallel_loop(0, KPAD // 16, unroll=1)
        def _(t):
            qi = selbuf[pl.ds(t * 16, 16)]
            qk = plsc.bitcast(gather_keys(qi), jnp.int32)

            def d(j, r):
                jj = j * 4
                for u in range(4):
                    dk = skey[jj + u]
                    di = sidx[jj + u]
                    beat = (dk > qk) | ((dk == qk) & (di < qi))
                    r = r + beat.astype(jnp.int32)
                return r
            rank = lax.fori_loop(0, KPAD // 4, d, zeros_i)

            score = plsc.bitcast(qk, jnp.float32)
            row = ((qi.astype(jnp.float32) + 0.5)
                   * jnp.float32(1.0 / 91.0)).astype(jnp.int32)
            row = jnp.minimum(row, 899)
            lab = qi - row * 91
            plsc.store_scatter(st_s, [rank], score)
            plsc.store_scatter(st_l, [rank], lab)
            for jb in range(4):
                bv = plsc.load_gather(boxv, [row * 4 + jb])
                plsc.store_scatter(st_b, [jb * KPAD + rank], bv)

        pltpu.sync_copy(st_s, scores_hbm.at[w])
        pltpu.sync_copy(st_l, labels_hbm.at[w])
        pltpu.sync_copy(st_b, boxout_hbm.at[w])

    return sc_topk


_sc_topk = _make_sc_topk()


def kernel(pred_logits, pred_boxes, target_sizes):
    del target_sizes
    b, q, cc = pred_logits.shape
    flat = pred_logits.reshape(b, q * cc)
    flat_pad = jnp.pad(flat, ((0, 0), (0, QCP - QC)),
                       constant_values=jnp.float32(-1e30))
    prob = _probs_tc(flat_pad.reshape(b * NROW, 128))
    boxes_flat = pred_boxes.reshape(b, q * 4)
    scores_p, labels_p, boxes_p = _sc_topk(prob, boxes_flat)
    scores = scores_p[:, :KOUT]
    labels = labels_p[:, :KOUT]
    boxes = boxes_p.reshape(b, 4, KPAD)[:, :, :KOUT].transpose(0, 2, 1)
    return scores, labels, boxes


# speedup vs baseline: 1.3813x; 1.0029x over previous
import functools

import jax
import jax.numpy as jnp
from jax import lax
from jax.experimental import pallas as pl
from jax.experimental.pallas import tpu as pltpu
from jax.experimental.pallas import tpu_sc as plsc

QC = 900 * 91
QCP = 81920
NV = QCP // 16
NB1 = 512
NB2 = 128
CAP = 6144
KOUT = 300
KPAD = 304


def _sigmoid_body(x_ref, o_ref):
    o_ref[...] = jax.nn.sigmoid(x_ref[...])


NROW = QCP // 128


def _probs_tc(flat_pad):
    rows = flat_pad.shape[0]
    return pl.pallas_call(
        _sigmoid_body,
        out_shape=jax.ShapeDtypeStruct((rows, 128), jnp.float32),
        grid=(4,),
        in_specs=[pl.BlockSpec((rows // 4, 128), lambda i: (i, 0))],
        out_specs=pl.BlockSpec((rows // 4, 128), lambda i: (i, 0)),
    )(flat_pad)


def _make_sc_topk():
    mesh = plsc.VectorSubcoreMesh(core_axis_name="c", subcore_axis_name="s")

    @functools.partial(
        pl.kernel,
        out_type=[
            jax.ShapeDtypeStruct((32, KPAD), jnp.float32),
            jax.ShapeDtypeStruct((32, KPAD), jnp.int32),
            jax.ShapeDtypeStruct((32, 4 * KPAD), jnp.float32),
        ],
        mesh=mesh,
        compiler_params=pltpu.CompilerParams(needs_layout_passes=False,
                                             use_tc_tiling_on_sc=True),
        scratch_types=[
            pltpu.VMEM((NROW, 128), jnp.float32),
            pltpu.VMEM((3600,), jnp.float32),
            pltpu.VMEM((16 * NB1,), jnp.int32),
            pltpu.VMEM((NB1 + 16,), jnp.int32),
            pltpu.VMEM((640,), jnp.int32),
            pltpu.VMEM((320,), jnp.int32),
            pltpu.VMEM((CAP + 16,), jnp.int32),
            pltpu.VMEM((KPAD,), jnp.float32),
            pltpu.VMEM((KPAD,), jnp.int32),
            pltpu.VMEM((4 * KPAD,), jnp.float32),
            pltpu.SMEM((KPAD,), jnp.int32),
            pltpu.SMEM((KPAD,), jnp.int32),
        ],
    )
    def sc_topk(prob_hbm, boxes_hbm, scores_hbm, labels_hbm, boxout_hbm,
                data, boxv, hist, tot, selbuf, eqbuf, candbuf,
                st_s, st_l, st_b, skey, sidx):
        w = lax.axis_index("s") * 2 + lax.axis_index("c")
        lane = jnp.arange(16, dtype=jnp.int32)
        zeros_i = jnp.zeros((16,), jnp.int32)
        ones_i = jnp.ones((16,), jnp.int32)

        pltpu.sync_copy(prob_hbm.at[pl.ds(w * NROW, NROW)], data)
        pltpu.sync_copy(boxes_hbm.at[w], boxv)

        def keys_at(i):
            return plsc.bitcast(data[i >> 3, pl.ds((i & 7) * 16, 16)],
                                jnp.uint32)

        def gather_keys(iv):
            pv = plsc.load_gather(data, [iv >> 7, iv & 127])
            return plsc.bitcast(pv, jnp.uint32)

        def zero_hist(nbins):
            @plsc.parallel_loop(0, nbins, unroll=8)
            def _(i):
                hist[pl.ds(i * 16, 16)] = zeros_i

        def find(nbins, target):
            ngr = nbins // 16

            @plsc.parallel_loop(0, ngr, unroll=2)
            def _(g):
                acc = zeros_i
                for l in range(16):
                    acc = acc + hist[pl.ds(l * nbins + g * 16, 16)]
                tot[pl.ds(g * 16, 16)] = acc
            tot[pl.ds(nbins, 16)] = zeros_i

            def s(gg, carry):
                g = ngr - 1 - gg
                tv = tot[pl.ds(g * 16, 16)]
                cs = plsc.cumsum(jnp.flip(tv, 0))
                tot[pl.ds(g * 16, 16)] = jnp.flip(cs, 0) + carry
                return carry + cs[15]
            lax.fori_loop(0, ngr, s, jnp.int32(0))

            def fb(g, best):
                sv = tot[pl.ds(g * 16, 16)]
                cand = jnp.where(sv >= target, g * 16 + lane, -1)
                return jnp.maximum(best, jnp.max(cand))
            b = lax.fori_loop(0, ngr, fb, jnp.int32(-1))
            above = tot[pl.ds(b + 1, 16)][0]
            return b, above

        zero_hist(NB1)
        laneoff1 = lane * NB1

        @plsc.parallel_loop(0, NV, unroll=16)
        def _(i):
            bn = (keys_at(i) >> 21).astype(jnp.int32)
            plsc.addupdate_scatter(hist, [laneoff1 + bn], ones_i)

        b1, ab1 = find(NB1, jnp.int32(KOUT))
        t2 = KOUT - ab1
        b1u = b1.astype(jnp.uint32)
        count_b1 = tot[pl.ds(b1, 16)][0] - ab1

        fifteen = jnp.full((16,), 15, jnp.int32)
        GRP = 8
        CAPG = CAP - (GRP * 16 - 16)

        gdn = lax.GatherDimensionNumbers(offset_dims=(),
                                         collapsed_slice_dims=(0,),
                                         start_index_map=(0,))

        def splat_last(cs):
            return lax.gather(cs, fifteen[:, None], gdn, (1,),
                              mode=lax.GatherScatterMode.PROMISE_IN_BOUNDS)

        def coll1(ii, carry):
            hi_v, c_v = carry
            gate = c_v < CAPG
            csh, csc, mh, mc, iv = [], [], [], [], []
            for s in range(GRP):
                i = ii * GRP + s
                top = keys_at(i) >> 21
                m_hi = top > b1u
                m_c = (top == b1u) & gate
                csh.append(plsc.cumsum(m_hi.astype(jnp.int32)))
                csc.append(plsc.cumsum(m_c.astype(jnp.int32)))
                mh.append(m_hi)
                mc.append(m_c)
                iv.append(i * 16 + lane)
            for s in range(GRP):
                plsc.store_scatter(selbuf, [hi_v + csh[s] - 1], iv[s],
                                   mask=mh[s])
                plsc.store_scatter(candbuf, [c_v + csc[s] - 1], iv[s],
                                   mask=mc[s])
                hi_v = hi_v + splat_last(csh[s])
                c_v = c_v + splat_last(csc[s])
            return hi_v, c_v
        hi_v, c_v = lax.fori_loop(0, NV // GRP, coll1, (zeros_i, zeros_i))
        c_hi, cur_c = hi_v[0], c_v[0]
        candbuf[pl.ds(cur_c, 16)] = jnp.full((16,), QC, jnp.int32)

        laneoff2 = lane * NB2

        def cand_keys(j):
            iv = candbuf[pl.ds(j * 16, 16)]
            return gather_keys(iv), iv

        def mid_small(_):
            ncv = (count_b1 + 15) // 16

            def histo_c(shift, pfx_shift, pfx):
                zero_hist(NB2)

                def hc(j, cc):
                    kv, _ = cand_keys(j)
                    m = (j * 16 + lane) < count_b1
                    if pfx_shift is not None:
                        m = m & ((kv >> pfx_shift) == pfx)
                    bn = ((kv >> shift) & 0x7F).astype(jnp.int32)
                    plsc.addupdate_scatter(hist, [laneoff2 + bn], ones_i,
                                           mask=m)
                    return cc
                lax.fori_loop(0, ncv, hc, 0)

            histo_c(14, None, None)
            b2, ab2 = find(NB2, t2)
            t3 = t2 - ab2
            p14 = (b1u << 7) | b2.astype(jnp.uint32)
            histo_c(7, 14, p14)
            b3, ab3 = find(NB2, t3)
            t4 = t3 - ab3
            p7 = (p14 << 7) | b3.astype(jnp.uint32)
            histo_c(0, 7, p7)
            b4, ab4 = find(NB2, t4)
            thr = (p7 << 7) | b4.astype(jnp.uint32)

            def cl(j, carry):
                cur_gt, cur_eq = carry
                kv, iv = cand_keys(j)
                valid = (j * 16 + lane) < count_b1
                m_gt = valid & (kv > thr)
                m_eq = (valid & (kv == thr)
                        & (jnp.broadcast_to(cur_eq, (16,)) < KPAD))
                plsc.store_compressed(selbuf.at[pl.ds(cur_gt, 16)], iv,
                                      mask=m_gt)
                plsc.store_compressed(eqbuf.at[pl.ds(cur_eq, 16)], iv,
                                      mask=m_eq)
                pg = plsc.all_reduce_population_count(m_gt)[0]
                pe = plsc.all_reduce_population_count(m_eq)[0]
                return cur_gt + pg, cur_eq + pe
            lax.fori_loop(0, ncv, cl, (c_hi, jnp.int32(0)))
            return ab2 + ab3 + ab4

        def mid_big(_):
            def histo_f(shift, pfx_shift, pfx):
                zero_hist(NB2)

                def hf(i, cc):
                    kv = keys_at(i)
                    m = (kv >> pfx_shift) == pfx
                    bn = ((kv >> shift) & 0x7F).astype(jnp.int32)
                    plsc.addupdate_scatter(hist, [laneoff2 + bn], ones_i,
                                           mask=m)
                    return cc
                lax.fori_loop(0, NV, hf, 0)

            histo_f(14, 21, b1u)
            b2, ab2 = find(NB2, t2)
            t3 = t2 - ab2
            p14 = (b1u << 7) | b2.astype(jnp.uint32)
            histo_f(7, 14, p14)
            b3, ab3 = find(NB2, t3)
            t4 = t3 - ab3
            p7 = (p14 << 7) | b3.astype(jnp.uint32)
            histo_f(0, 7, p7)
            b4, ab4 = find(NB2, t4)
            thr = (p7 << 7) | b4.astype(jnp.uint32)

            def cl(i, carry):
                cur_gt, cur_eq = carry
                kv = keys_at(i)
                iv = i * 16 + lane
                in_b1 = (kv >> 21) == b1u
                m_gt = in_b1 & (kv > thr)
                m_eq = (in_b1 & (kv == thr)
                        & (jnp.broadcast_to(cur_eq, (16,)) < KPAD))
                plsc.store_compressed(selbuf.at[pl.ds(cur_gt, 16)], iv,
                                      mask=m_gt)
                plsc.store_compressed(eqbuf.at[pl.ds(cur_eq, 16)], iv,
                                      mask=m_eq)
                pg = plsc.all_reduce_population_count(m_gt)[0]
                pe = plsc.all_reduce_population_count(m_eq)[0]
                return cur_gt + pg, cur_eq + pe
            lax.fori_loop(0, NV, cl, (c_hi, jnp.int32(0)))
            return ab2 + ab3 + ab4

        ab234 = lax.cond(count_b1 <= CAPG, mid_small, mid_big, 0)
        c = ab1 + ab234

        @plsc.parallel_loop(0, KPAD // 16, unroll=1)
        def _(t):
            selbuf[pl.ds(c + t * 16, 16)] = eqbuf[pl.ds(t * 16, 16)]
        selbuf[pl.ds(KOUT, 16)] = QC + lane

        @plsc.parallel_loop(0, KPAD // 16, unroll=1)
        def _(t):
            iv = selbuf[pl.ds(t * 16, 16)]
            kv = plsc.bitcast(gather_keys(iv), jnp.int32)
            for l in range(16):
                skey[t * 16 + l] = kv[l]
                sidx[t * 16 + l] = iv[l]

        @plsc.parallel_loop(0, KPAD // 16, unroll=1)
        def _(t):
            qi = selbuf[pl.ds(t * 16, 16)]
            qk = plsc.bitcast(gather_keys(qi), jnp.int32)

            def d(j, r):
                jj = j * 4
                for u in range(4):
                    dk = skey[jj + u]
                    di = sidx[jj + u]
                    beat = (dk > qk) | ((dk == qk) & (di < qi))
                    r = r + beat.astype(jnp.int32)
                return r
            rank = lax.fori_loop(0, KPAD // 4, d, zeros_i)

            score = plsc.bitcast(qk, jnp.float32)
            row = ((qi.astype(jnp.float32) + 0.5)
                   * jnp.float32(1.0 / 91.0)).astype(jnp.int32)
            row = jnp.minimum(row, 899)
            lab = qi - row * 91
            plsc.store_scatter(st_s, [rank], score)
            plsc.store_scatter(st_l, [rank], lab)
            for jb in range(4):
                bv = plsc.load_gather(boxv, [row * 4 + jb])
                plsc.store_scatter(st_b, [jb * KPAD + rank], bv)

        pltpu.sync_copy(st_s, scores_hbm.at[w])
        pltpu.sync_copy(st_l, labels_hbm.at[w])
        pltpu.sync_copy(st_b, boxout_hbm.at[w])

    return sc_topk


_sc_topk = _make_sc_topk()


def kernel(pred_logits, pred_boxes, target_sizes):
    del target_sizes
    b, q, cc = pred_logits.shape
    flat = pred_logits.reshape(b, q * cc)
    flat_pad = jnp.pad(flat, ((0, 0), (0, QCP - QC)),
                       constant_values=jnp.float32(-1e30))
    prob = _probs_tc(flat_pad.reshape(b * NROW, 128))
    boxes_flat = pred_boxes.reshape(b, q * 4)
    scores_p, labels_p, boxes_p = _sc_topk(prob, boxes_flat)
    scores = scores_p[:, :KOUT]
    labels = labels_p[:, :KOUT]
    boxes = boxes_p.reshape(b, 4, KPAD)[:, :, :KOUT].transpose(0, 2, 1)
    return scores, labels, boxes


# speedup vs baseline: 1.4074x; 1.0189x over previous
import functools

import jax
import jax.numpy as jnp
from jax import lax
from jax.experimental import pallas as pl
from jax.experimental.pallas import tpu as pltpu
from jax.experimental.pallas import tpu_sc as plsc

QC = 900 * 91
QCP = 81920
NV = QCP // 16
NB1 = 512
NB2 = 128
CAP = 6144
KOUT = 300
KPAD = 304


def _sigmoid_body(x_ref, o_ref):
    o_ref[...] = jax.nn.sigmoid(x_ref[...])


NROW = QCP // 128


def _probs_tc(flat_pad):
    rows = flat_pad.shape[0]
    return pl.pallas_call(
        _sigmoid_body,
        out_shape=jax.ShapeDtypeStruct((rows, 128), jnp.float32),
        grid=(4,),
        in_specs=[pl.BlockSpec((rows // 4, 128), lambda i: (i, 0))],
        out_specs=pl.BlockSpec((rows // 4, 128), lambda i: (i, 0)),
    )(flat_pad)


def _make_sc_topk():
    mesh = plsc.VectorSubcoreMesh(core_axis_name="c", subcore_axis_name="s")

    @functools.partial(
        pl.kernel,
        out_type=[
            jax.ShapeDtypeStruct((32, KPAD), jnp.float32),
            jax.ShapeDtypeStruct((32, KPAD), jnp.int32),
            jax.ShapeDtypeStruct((32, 4 * KPAD), jnp.float32),
        ],
        mesh=mesh,
        compiler_params=pltpu.CompilerParams(needs_layout_passes=False,
                                             use_tc_tiling_on_sc=True),
        scratch_types=[
            pltpu.VMEM((QCP,), jnp.float32),
            pltpu.VMEM((3600,), jnp.float32),
            pltpu.VMEM((16 * NB1,), jnp.int32),
            pltpu.VMEM((NB1 + 16,), jnp.int32),
            pltpu.VMEM((640,), jnp.int32),
            pltpu.VMEM((320,), jnp.int32),
            pltpu.VMEM((CAP + 16,), jnp.int32),
            pltpu.VMEM((KPAD,), jnp.float32),
            pltpu.VMEM((KPAD,), jnp.int32),
            pltpu.VMEM((4 * KPAD,), jnp.float32),
            pltpu.SMEM((KPAD,), jnp.int32),
            pltpu.SMEM((KPAD,), jnp.int32),
        ],
    )
    def sc_topk(prob_hbm, boxes_hbm, scores_hbm, labels_hbm, boxout_hbm,
                data, boxv, hist, tot, selbuf, eqbuf, candbuf,
                st_s, st_l, st_b, skey, sidx):
        w = lax.axis_index("s") * 2 + lax.axis_index("c")
        lane = jnp.arange(16, dtype=jnp.int32)
        zeros_i = jnp.zeros((16,), jnp.int32)
        ones_i = jnp.ones((16,), jnp.int32)

        pltpu.sync_copy(prob_hbm.at[pl.ds(w * QCP, QCP)], data)
        pltpu.sync_copy(boxes_hbm.at[w], boxv)

        def keys_at(i):
            return plsc.bitcast(data[pl.ds(i * 16, 16)], jnp.uint32)

        def gather_keys(iv):
            return plsc.bitcast(plsc.load_gather(data, [iv]), jnp.uint32)

        def zero_hist(nbins):
            @plsc.parallel_loop(0, nbins, unroll=8)
            def _(i):
                hist[pl.ds(i * 16, 16)] = zeros_i

        def find(nbins, target):
            ngr = nbins // 16

            @plsc.parallel_loop(0, ngr, unroll=2)
            def _(g):
                acc = zeros_i
                for l in range(16):
                    acc = acc + hist[pl.ds(l * nbins + g * 16, 16)]
                tot[pl.ds(g * 16, 16)] = acc
            tot[pl.ds(nbins, 16)] = zeros_i

            def s(gg, carry):
                g = ngr - 1 - gg
                tv = tot[pl.ds(g * 16, 16)]
                cs = plsc.cumsum(jnp.flip(tv, 0))
                tot[pl.ds(g * 16, 16)] = jnp.flip(cs, 0) + carry
                return carry + cs[15]
            lax.fori_loop(0, ngr, s, jnp.int32(0))

            def fb(g, best):
                sv = tot[pl.ds(g * 16, 16)]
                cand = jnp.where(sv >= target, g * 16 + lane, -1)
                return jnp.maximum(best, jnp.max(cand))
            b = lax.fori_loop(0, ngr, fb, jnp.int32(-1))
            above = tot[pl.ds(b + 1, 16)][0]
            return b, above

        zero_hist(NB1)
        laneoff1 = lane * NB1

        @plsc.parallel_loop(0, NV, unroll=16)
        def _(i):
            bn = (keys_at(i) >> 21).astype(jnp.int32)
            plsc.addupdate_scatter(hist, [laneoff1 + bn], ones_i)

        b1, ab1 = find(NB1, jnp.int32(KOUT))
        t2 = KOUT - ab1
        b1u = b1.astype(jnp.uint32)
        count_b1 = tot[pl.ds(b1, 16)][0] - ab1

        fifteen = jnp.full((16,), 15, jnp.int32)
        GRP = 8
        CAPG = CAP - (GRP * 16 - 16)

        gdn = lax.GatherDimensionNumbers(offset_dims=(),
                                         collapsed_slice_dims=(0,),
                                         start_index_map=(0,))

        def splat_last(cs):
            return lax.gather(cs, fifteen[:, None], gdn, (1,),
                              mode=lax.GatherScatterMode.PROMISE_IN_BOUNDS)

        def coll1(ii, carry):
            hi_v, c_v = carry
            gate = c_v < CAPG
            csh, csc, mh, mc, iv = [], [], [], [], []
            for s in range(GRP):
                i = ii * GRP + s
                top = keys_at(i) >> 21
                m_hi = top > b1u
                m_c = (top == b1u) & gate
                csh.append(plsc.cumsum(m_hi.astype(jnp.int32)))
                csc.append(plsc.cumsum(m_c.astype(jnp.int32)))
                mh.append(m_hi)
                mc.append(m_c)
                iv.append(i * 16 + lane)
            for s in range(GRP):
                plsc.store_scatter(selbuf, [hi_v + csh[s] - 1], iv[s],
                                   mask=mh[s])
                plsc.store_scatter(candbuf, [c_v + csc[s] - 1], iv[s],
                                   mask=mc[s])
                hi_v = hi_v + splat_last(csh[s])
                c_v = c_v + splat_last(csc[s])
            return hi_v, c_v
        hi_v, c_v = lax.fori_loop(0, NV // GRP, coll1, (zeros_i, zeros_i))
        c_hi, cur_c = hi_v[0], c_v[0]
        candbuf[pl.ds(cur_c, 16)] = jnp.full((16,), QC, jnp.int32)

        laneoff2 = lane * NB2

        def cand_keys(j):
            iv = candbuf[pl.ds(j * 16, 16)]
            return gather_keys(iv), iv

        def mid_small(_):
            ncv = (count_b1 + 15) // 16

            def histo_c(shift, pfx_shift, pfx):
                zero_hist(NB2)

                def hc(j, cc):
                    kv, _ = cand_keys(j)
                    m = (j * 16 + lane) < count_b1
                    if pfx_shift is not None:
                        m = m & ((kv >> pfx_shift) == pfx)
                    bn = ((kv >> shift) & 0x7F).astype(jnp.int32)
                    plsc.addupdate_scatter(hist, [laneoff2 + bn], ones_i,
                                           mask=m)
                    return cc
                lax.fori_loop(0, ncv, hc, 0)

            histo_c(14, None, None)
            b2, ab2 = find(NB2, t2)
            t3 = t2 - ab2
            p14 = (b1u << 7) | b2.astype(jnp.uint32)
            histo_c(7, 14, p14)
            b3, ab3 = find(NB2, t3)
            t4 = t3 - ab3
            p7 = (p14 << 7) | b3.astype(jnp.uint32)
            histo_c(0, 7, p7)
            b4, ab4 = find(NB2, t4)
            thr = (p7 << 7) | b4.astype(jnp.uint32)

            def cl(j, carry):
                cur_gt, cur_eq = carry
                kv, iv = cand_keys(j)
                valid = (j * 16 + lane) < count_b1
                m_gt = valid & (kv > thr)
                m_eq = (valid & (kv == thr)
                        & (jnp.broadcast_to(cur_eq, (16,)) < KPAD))
                plsc.store_compressed(selbuf.at[pl.ds(cur_gt, 16)], iv,
                                      mask=m_gt)
                plsc.store_compressed(eqbuf.at[pl.ds(cur_eq, 16)], iv,
                                      mask=m_eq)
                pg = plsc.all_reduce_population_count(m_gt)[0]
                pe = plsc.all_reduce_population_count(m_eq)[0]
                return cur_gt + pg, cur_eq + pe
            lax.fori_loop(0, ncv, cl, (c_hi, jnp.int32(0)))
            return ab2 + ab3 + ab4

        def mid_big(_):
            def histo_f(shift, pfx_shift, pfx):
                zero_hist(NB2)

                def hf(i, cc):
                    kv = keys_at(i)
                    m = (kv >> pfx_shift) == pfx
                    bn = ((kv >> shift) & 0x7F).astype(jnp.int32)
                    plsc.addupdate_scatter(hist, [laneoff2 + bn], ones_i,
                                           mask=m)
                    return cc
                lax.fori_loop(0, NV, hf, 0)

            histo_f(14, 21, b1u)
            b2, ab2 = find(NB2, t2)
            t3 = t2 - ab2
            p14 = (b1u << 7) | b2.astype(jnp.uint32)
            histo_f(7, 14, p14)
            b3, ab3 = find(NB2, t3)
            t4 = t3 - ab3
            p7 = (p14 << 7) | b3.astype(jnp.uint32)
            histo_f(0, 7, p7)
            b4, ab4 = find(NB2, t4)
            thr = (p7 << 7) | b4.astype(jnp.uint32)

            def cl(i, carry):
                cur_gt, cur_eq = carry
                kv = keys_at(i)
                iv = i * 16 + lane
                in_b1 = (kv >> 21) == b1u
                m_gt = in_b1 & (kv > thr)
                m_eq = (in_b1 & (kv == thr)
                        & (jnp.broadcast_to(cur_eq, (16,)) < KPAD))
                plsc.store_compressed(selbuf.at[pl.ds(cur_gt, 16)], iv,
                                      mask=m_gt)
                plsc.store_compressed(eqbuf.at[pl.ds(cur_eq, 16)], iv,
                                      mask=m_eq)
                pg = plsc.all_reduce_population_count(m_gt)[0]
                pe = plsc.all_reduce_population_count(m_eq)[0]
                return cur_gt + pg, cur_eq + pe
            lax.fori_loop(0, NV, cl, (c_hi, jnp.int32(0)))
            return ab2 + ab3 + ab4

        ab234 = lax.cond(count_b1 <= CAPG, mid_small, mid_big, 0)
        c = ab1 + ab234

        @plsc.parallel_loop(0, KPAD // 16, unroll=1)
        def _(t):
            selbuf[pl.ds(c + t * 16, 16)] = eqbuf[pl.ds(t * 16, 16)]
        selbuf[pl.ds(KOUT, 16)] = QC + lane

        @plsc.parallel_loop(0, KPAD // 16, unroll=1)
        def _(t):
            iv = selbuf[pl.ds(t * 16, 16)]
            kv = plsc.bitcast(gather_keys(iv), jnp.int32)
            for l in range(16):
                skey[t * 16 + l] = kv[l]
                sidx[t * 16 + l] = iv[l]

        @plsc.parallel_loop(0, KPAD // 16, unroll=1)
        def _(t):
            qi = selbuf[pl.ds(t * 16, 16)]
            qk = plsc.bitcast(gather_keys(qi), jnp.int32)

            def d(j, r):
                jj = j * 4
                for u in range(4):
                    dk = skey[jj + u]
                    di = sidx[jj + u]
                    beat = (dk > qk) | ((dk == qk) & (di < qi))
                    r = r + beat.astype(jnp.int32)
                return r
            rank = lax.fori_loop(0, KPAD // 4, d, zeros_i)

            score = plsc.bitcast(qk, jnp.float32)
            row = ((qi.astype(jnp.float32) + 0.5)
                   * jnp.float32(1.0 / 91.0)).astype(jnp.int32)
            row = jnp.minimum(row, 899)
            lab = qi - row * 91
            plsc.store_scatter(st_s, [rank], score)
            plsc.store_scatter(st_l, [rank], lab)
            for jb in range(4):
                bv = plsc.load_gather(boxv, [row * 4 + jb])
                plsc.store_scatter(st_b, [jb * KPAD + rank], bv)

        pltpu.sync_copy(st_s, scores_hbm.at[w])
        pltpu.sync_copy(st_l, labels_hbm.at[w])
        pltpu.sync_copy(st_b, boxout_hbm.at[w])

    return sc_topk


_sc_topk = _make_sc_topk()


def kernel(pred_logits, pred_boxes, target_sizes):
    del target_sizes
    b, q, cc = pred_logits.shape
    flat = pred_logits.reshape(b, q * cc)
    flat_pad = jnp.pad(flat, ((0, 0), (0, QCP - QC)),
                       constant_values=jnp.float32(-1e30))
    prob = _probs_tc(flat_pad.reshape(b * NROW, 128)).reshape(b * QCP)
    boxes_flat = pred_boxes.reshape(b, q * 4)
    scores_p, labels_p, boxes_p = _sc_topk(prob, boxes_flat)
    scores = scores_p[:, :KOUT]
    labels = labels_p[:, :KOUT]
    boxes = boxes_p.reshape(b, 4, KPAD)[:, :, :KOUT].transpose(0, 2, 1)
    return scores, labels, boxes
